# Initial kernel scaffold; baseline (speedup 1.0000x reference)
import functools

import jax
import jax.numpy as jnp
from jax import lax
from jax.experimental import pallas as pl
from jax.experimental.pallas import tpu as pltpu
from jax.experimental.pallas import tpu_sc as plsc

V = 1000000
D = 64
B = 16384
K = 10

NC = 2
NS = 16
NW = NC * NS
BPW = B // NW
CHUNK = 128
NCH = BPW // CHUNK


def _sc_gather_body(center_hbm, pos_hbm, negf_hbm, in_embed_hbm, out_embed_hbm,
                    v_out, upos_out, uneg_out, idx_c, idx_n, rows, sem):
    wid = lax.axis_index("s") * NC + lax.axis_index("c")
    base = wid * BPW

    def gather_group(table_hbm, idx_ref, idx_off, out_hbm, out_off):
        copies = []
        for c in range(NCH):
            copies.append(pltpu.async_copy(
                table_hbm.at[idx_ref.at[pl.ds(idx_off + c * CHUNK, CHUNK)]],
                rows.at[pl.ds(c * CHUNK, CHUNK), :],
                sem))
        for cp in copies:
            cp.wait()
        pltpu.sync_copy(rows, out_hbm.at[pl.ds(out_off, BPW)])

    pltpu.sync_copy(center_hbm.at[pl.ds(base, BPW)], idx_c)
    gather_group(in_embed_hbm, idx_c, 0, v_out, base)
    pltpu.sync_copy(pos_hbm.at[pl.ds(base, BPW)], idx_c)
    gather_group(out_embed_hbm, idx_c, 0, upos_out, base)
    pltpu.sync_copy(negf_hbm.at[pl.ds(base * K, BPW * K)], idx_n)
    for g in range(K):
        gather_group(out_embed_hbm, idx_n, g * BPW, uneg_out, base * K + g * BPW)


def _make_sc_gather():
    mesh = plsc.VectorSubcoreMesh(core_axis_name="c", subcore_axis_name="s")
    return functools.partial(
        pl.kernel,
        out_type=(
            jax.ShapeDtypeStruct((B, D), jnp.float32),
            jax.ShapeDtypeStruct((B, D), jnp.float32),
            jax.ShapeDtypeStruct((B * K, D), jnp.float32),
        ),
        mesh=mesh,
        scratch_types=[
            pltpu.VMEM((BPW,), jnp.int32),
            pltpu.VMEM((BPW * K,), jnp.int32),
            pltpu.VMEM((BPW, D), jnp.float32),
            pltpu.SemaphoreType.DMA,
        ],
    )(_sc_gather_body)


_BB = 1024


def _tc_loss_body(v_ref, up_ref, un_ref, out_ref):
    i = pl.program_id(0)
    vb = v_ref[...]
    pos_score = jnp.sum(vb * up_ref[...], axis=1, keepdims=True)
    acc = jnp.sum(jnp.log(1.0 / (1.0 + jnp.exp(-pos_score)) + 1e-9)) / B
    neg_acc = jnp.float32(0.0)
    for k in range(K):
        ns = jnp.sum(vb * un_ref[:, k, :], axis=1, keepdims=True)
        neg_acc += jnp.sum(jnp.log(1.0 / (1.0 + jnp.exp(ns)) + 1e-9))
    acc += neg_acc / (B * K)

    @pl.when(i == 0)
    def _():
        out_ref[0, 0] = jnp.float32(0.0)

    out_ref[0, 0] += -acc


def _tc_loss(v, u_pos, u_neg):
    return pl.pallas_call(
        _tc_loss_body,
        grid=(B // _BB,),
        in_specs=[
            pl.BlockSpec((_BB, D), lambda i: (i, 0)),
            pl.BlockSpec((_BB, D), lambda i: (i, 0)),
            pl.BlockSpec((_BB, K, D), lambda i: (i, 0, 0)),
        ],
        out_specs=pl.BlockSpec(memory_space=pltpu.SMEM),
        out_shape=jax.ShapeDtypeStruct((1, 1), jnp.float32),
    )(v, u_pos, u_neg)


def kernel(center, pos, neg, in_embed, out_embed):
    center = center.astype(jnp.int32)
    pos = pos.astype(jnp.int32)
    neg_flat = neg.astype(jnp.int32).reshape(B * K)
    v, u_pos, u_neg = _make_sc_gather()(center, pos, neg_flat, in_embed, out_embed)
    loss = _tc_loss(v, u_pos, u_neg.reshape(B, K, D))
    return loss[0, 0]


# speedup vs baseline: 2.5921x; 2.5921x over previous
import functools

import jax
import jax.numpy as jnp
from jax import lax
from jax.experimental import pallas as pl
from jax.experimental.pallas import tpu as pltpu
from jax.experimental.pallas import tpu_sc as plsc

V = 1000000
D = 64
B = 16384
K = 10

NC = 2
NS = 16
NW = NC * NS
BPW = B // NW
CHUNK = 128
NCH = BPW // CHUNK


def _sc_gather_body(center_hbm, pos_hbm, negf_hbm, in_embed_hbm, out_embed_hbm,
                    v_out, upos_out, uneg_out, idx_c, idx_n, rows, sem):
    wid = lax.axis_index("s") * NC + lax.axis_index("c")
    base = wid * BPW

    def gather_group(table_hbm, idx_ref, idx_off, out_hbm, out_off):
        copies = []
        for c in range(NCH):
            copies.append(pltpu.async_copy(
                table_hbm.at[idx_ref.at[pl.ds(idx_off + c * CHUNK, CHUNK)]],
                rows.at[pl.ds(c * CHUNK, CHUNK), :],
                sem))
        for cp in copies:
            cp.wait()
        pltpu.sync_copy(rows, out_hbm.at[pl.ds(out_off, BPW)])

    pltpu.sync_copy(center_hbm.at[pl.ds(base, BPW)], idx_c)
    gather_group(in_embed_hbm, idx_c, 0, v_out, base)
    pltpu.sync_copy(pos_hbm.at[pl.ds(base, BPW)], idx_c)
    gather_group(out_embed_hbm, idx_c, 0, upos_out, base)
    pltpu.sync_copy(negf_hbm.at[pl.ds(base * K, BPW * K)], idx_n)
    for g in range(K):
        gather_group(out_embed_hbm, idx_n, g * BPW, uneg_out, base * K + g * BPW)


def _make_sc_gather():
    mesh = plsc.VectorSubcoreMesh(core_axis_name="c", subcore_axis_name="s")
    return functools.partial(
        pl.kernel,
        out_type=(
            jax.ShapeDtypeStruct((B, D), jnp.float32),
            jax.ShapeDtypeStruct((B, D), jnp.float32),
            jax.ShapeDtypeStruct((B * K, D), jnp.float32),
        ),
        mesh=mesh,
        scratch_types=[
            pltpu.VMEM((BPW,), jnp.int32),
            pltpu.VMEM((BPW * K,), jnp.int32),
            pltpu.VMEM((BPW, D), jnp.float32),
            pltpu.SemaphoreType.DMA,
        ],
        compiler_params=pltpu.CompilerParams(use_tc_tiling_on_sc=False),
    )(_sc_gather_body)


_BB = 1024


def _tc_loss_body(v_ref, up_ref, un_ref, out_ref):
    i = pl.program_id(0)
    vb = v_ref[...]
    pos_score = jnp.sum(vb * up_ref[...], axis=1, keepdims=True)
    acc = jnp.sum(jnp.log(1.0 / (1.0 + jnp.exp(-pos_score)) + 1e-9)) / B
    neg_acc = jnp.float32(0.0)
    for k in range(K):
        ns = jnp.sum(vb * un_ref[:, k, :], axis=1, keepdims=True)
        neg_acc += jnp.sum(jnp.log(1.0 / (1.0 + jnp.exp(ns)) + 1e-9))
    acc += neg_acc / (B * K)

    @pl.when(i == 0)
    def _():
        out_ref[0, 0] = jnp.float32(0.0)

    out_ref[0, 0] += -acc


def _tc_loss(v, u_pos, u_neg):
    return pl.pallas_call(
        _tc_loss_body,
        grid=(B // _BB,),
        in_specs=[
            pl.BlockSpec((_BB, D), lambda i: (i, 0)),
            pl.BlockSpec((_BB, D), lambda i: (i, 0)),
            pl.BlockSpec((_BB, K, D), lambda i: (i, 0, 0)),
        ],
        out_specs=pl.BlockSpec(memory_space=pltpu.SMEM),
        out_shape=jax.ShapeDtypeStruct((1, 1), jnp.float32),
    )(v, u_pos, u_neg)


def kernel(center, pos, neg, in_embed, out_embed):
    center = center.astype(jnp.int32)
    pos = pos.astype(jnp.int32)
    neg_flat = neg.astype(jnp.int32).reshape(B * K)
    v, u_pos, u_neg = _make_sc_gather()(center, pos, neg_flat, in_embed, out_embed)
    loss = _tc_loss(v, u_pos, u_neg.reshape(B, K, D))
    return loss[0, 0]


# speedup vs baseline: 2.5950x; 1.0011x over previous
import functools

import jax
import jax.numpy as jnp
from jax import lax
from jax.experimental import pallas as pl
from jax.experimental.pallas import tpu as pltpu
from jax.experimental.pallas import tpu_sc as plsc

V = 1000000
D = 64
B = 16384
K = 10

NC = 2
NS = 16
NW = NC * NS
BPW = B // NW
SCR = 128
SUP = BPW // SCR
NG = SCR // 16
L = 16


def _sc_body(center_hbm, pos_hbm, negf_hbm, in_emb, out_emb,
             ps_out, ns_out,
             idx_c, idx_p, idx_n, v_rows, up_rows, un_rows, ps_buf, ns_buf, sem):
    wid = lax.axis_index("s") * NC + lax.axis_index("c")
    base = wid * BPW

    pltpu.sync_copy(center_hbm.at[pl.ds(base, BPW)], idx_c)
    pltpu.sync_copy(pos_hbm.at[pl.ds(base, BPW)], idx_p)
    pltpu.sync_copy(negf_hbm.at[pl.ds(base * K, BPW * K)], idx_n)

    iota = lax.broadcasted_iota(jnp.int32, (L,), 0)
    zero = jnp.zeros((L,), jnp.float32)

    def super_chunk(c, carry):
        cps = [
            pltpu.async_copy(in_emb.at[idx_c.at[pl.ds(c * SCR, SCR)]], v_rows, sem),
            pltpu.async_copy(out_emb.at[idx_p.at[pl.ds(c * SCR, SCR)]], up_rows, sem),
        ]
        for q in range(K):
            cps.append(pltpu.async_copy(
                out_emb.at[idx_n.at[pl.ds(c * SCR * K + q * SCR, SCR)]],
                un_rows.at[pl.ds(q * SCR, SCR), :], sem))
        for cp in cps:
            cp.wait()

        def group(g, carry2):
            rowv = iota + g * L
            rvK = rowv * K
            un_rowvs = [rvK + q for q in range(K)]
            ps_acc = zero
            accs = [zero] * K
            for d in range(D):
                colv = jnp.full((L,), d, jnp.int32)
                vcol = plsc.load_gather(v_rows, [rowv, colv])
                upcol = plsc.load_gather(up_rows, [rowv, colv])
                ps_acc = ps_acc + vcol * upcol
                for q in range(K):
                    ucol = plsc.load_gather(un_rows, [un_rowvs[q], colv])
                    accs[q] = accs[q] - vcol * ucol
            off = c * SCR + g * L
            ps_buf[pl.ds(off, L)] = ps_acc
            for q in range(K):
                ns_buf[pl.ds(c * SCR * K + q * SCR + g * L, L)] = accs[q]
            return carry2

        return lax.fori_loop(0, NG, group, carry)

    lax.fori_loop(0, SUP, super_chunk, 0)

    pltpu.sync_copy(ps_buf, ps_out.at[pl.ds(base, BPW)])
    pltpu.sync_copy(ns_buf, ns_out.at[pl.ds(base * K, BPW * K)])


def _make_sc_scores():
    mesh = plsc.VectorSubcoreMesh(core_axis_name="c", subcore_axis_name="s")
    return functools.partial(
        pl.kernel,
        out_type=(
            jax.ShapeDtypeStruct((B,), jnp.float32),
            jax.ShapeDtypeStruct((B * K,), jnp.float32),
        ),
        mesh=mesh,
        scratch_types=[
            pltpu.VMEM((BPW,), jnp.int32),
            pltpu.VMEM((BPW,), jnp.int32),
            pltpu.VMEM((BPW * K,), jnp.int32),
            pltpu.VMEM((SCR, D), jnp.float32),
            pltpu.VMEM((SCR, D), jnp.float32),
            pltpu.VMEM((SCR * K, D), jnp.float32),
            pltpu.VMEM((BPW,), jnp.float32),
            pltpu.VMEM((BPW * K,), jnp.float32),
            pltpu.SemaphoreType.DMA,
        ],
        compiler_params=pltpu.CompilerParams(use_tc_tiling_on_sc=False,
                                             needs_layout_passes=False),
    )(_sc_body)


def _tc_loss_body(ps_ref, ns_ref, out_ref):
    lp = jnp.log(1.0 / (1.0 + jnp.exp(-ps_ref[...])) + 1e-9)
    ln = jnp.log(1.0 / (1.0 + jnp.exp(-ns_ref[...])) + 1e-9)
    out_ref[0, 0] = -(jnp.sum(lp) / B + jnp.sum(ln) / (B * K))


def _tc_loss(ps, ns):
    return pl.pallas_call(
        _tc_loss_body,
        out_specs=pl.BlockSpec(memory_space=pltpu.SMEM),
        out_shape=jax.ShapeDtypeStruct((1, 1), jnp.float32),
    )(ps.reshape(B // 128, 128), ns.reshape(B * K // 128, 128))


def kernel(center, pos, neg, in_embed, out_embed):
    center = center.astype(jnp.int32)
    pos = pos.astype(jnp.int32)
    neg_flat = neg.astype(jnp.int32).reshape(B * K)
    ps, ns = _make_sc_scores()(center, pos, neg_flat, in_embed, out_embed)
    return _tc_loss(ps, ns)[0, 0]


# speedup vs baseline: 2.8762x; 1.1083x over previous
import functools

import jax
import jax.numpy as jnp
from jax import lax
from jax.experimental import pallas as pl
from jax.experimental.pallas import tpu as pltpu
from jax.experimental.pallas import tpu_sc as plsc

V = 1000000
D = 64
B = 16384
K = 10

NC = 2
NS = 16
NW = NC * NS
BPW = B // NW
SCR = 64
SUP = BPW // SCR
L = 16
NCHK = D // L


def _sc_body(center_hbm, pos_hbm, negf_hbm, in_emb, out_emb,
             pacc_out, nacc_out,
             idx_c, idx_p, idx_n, v_rows, up_rows, un_rows,
             pacc_buf, nacc_buf, sem):
    wid = lax.axis_index("s") * NC + lax.axis_index("c")
    base = wid * BPW

    pltpu.sync_copy(center_hbm.at[pl.ds(base, BPW)], idx_c)
    pltpu.sync_copy(pos_hbm.at[pl.ds(base, BPW)], idx_p)
    pltpu.sync_copy(negf_hbm.at[pl.ds(base * K, BPW * K)], idx_n)

    def super_chunk(c, carry):
        cps = [
            pltpu.async_copy(in_emb.at[idx_c.at[pl.ds(c * SCR, SCR)]], v_rows, sem),
            pltpu.async_copy(out_emb.at[idx_p.at[pl.ds(c * SCR, SCR)]], up_rows, sem),
        ]
        for q in range(K):
            cps.append(pltpu.async_copy(
                out_emb.at[idx_n.at[pl.ds(c * SCR * K + q * SCR, SCR)]],
                un_rows.at[pl.ds(q * SCR, SCR), :], sem))
        for cp in cps:
            cp.wait()

        def row(r, carry2):
            vc = [v_rows[r, pl.ds(ch * L, L)] for ch in range(NCHK)]
            uc = [up_rows[r, pl.ds(ch * L, L)] for ch in range(NCHK)]
            p = vc[0] * uc[0]
            for ch in range(1, NCHK):
                p = p + vc[ch] * uc[ch]
            pacc_buf[pl.ds(r * L, L)] = p
            rK = r * K
            for q in range(K):
                nc_ = [un_rows[rK + q, pl.ds(ch * L, L)] for ch in range(NCHK)]
                n = vc[0] * nc_[0]
                for ch in range(1, NCHK):
                    n = n + vc[ch] * nc_[ch]
                nacc_buf[pl.ds((rK + q) * L, L)] = -n
            return carry2

        lax.fori_loop(0, SCR, row, carry)
        pltpu.sync_copy(pacc_buf,
                        pacc_out.at[pl.ds((base + c * SCR) * L, SCR * L)])
        pltpu.sync_copy(nacc_buf,
                        nacc_out.at[pl.ds((base * K + c * SCR * K) * L, SCR * K * L)])
        return carry

    lax.fori_loop(0, SUP, super_chunk, 0)


def _make_sc_scores():
    mesh = plsc.VectorSubcoreMesh(core_axis_name="c", subcore_axis_name="s")
    return functools.partial(
        pl.kernel,
        out_type=(
            jax.ShapeDtypeStruct((B * L,), jnp.float32),
            jax.ShapeDtypeStruct((B * K * L,), jnp.float32),
        ),
        mesh=mesh,
        scratch_types=[
            pltpu.VMEM((BPW,), jnp.int32),
            pltpu.VMEM((BPW,), jnp.int32),
            pltpu.VMEM((BPW * K,), jnp.int32),
            pltpu.VMEM((SCR, D), jnp.float32),
            pltpu.VMEM((SCR, D), jnp.float32),
            pltpu.VMEM((SCR * K, D), jnp.float32),
            pltpu.VMEM((SCR * L,), jnp.float32),
            pltpu.VMEM((SCR * K * L,), jnp.float32),
            pltpu.SemaphoreType.DMA,
        ],
        compiler_params=pltpu.CompilerParams(use_tc_tiling_on_sc=False,
                                             needs_layout_passes=False),
    )(_sc_body)


def _tc_loss_body(pa_ref, na_ref, out_ref):
    i = lax.broadcasted_iota(jnp.int32, (128, 128), 0)
    j = lax.broadcasted_iota(jnp.int32, (128, 128), 1)
    S = jnp.where(i // L == j, 1.0, 0.0).astype(jnp.float32)
    lane = lax.broadcasted_iota(jnp.int32, (1, 128), 1)
    mask = lane < (128 // L)

    ps = jnp.dot(pa_ref[...], S, preferred_element_type=jnp.float32)
    lp = jnp.where(mask, jnp.log(1.0 / (1.0 + jnp.exp(-ps)) + 1e-9), 0.0)
    ns = jnp.dot(na_ref[...], S, preferred_element_type=jnp.float32)
    ln = jnp.where(mask, jnp.log(1.0 / (1.0 + jnp.exp(-ns)) + 1e-9), 0.0)
    out_ref[0, 0] = -(jnp.sum(lp) / B + jnp.sum(ln) / (B * K))


def _tc_loss(pacc, nacc):
    return pl.pallas_call(
        _tc_loss_body,
        out_specs=pl.BlockSpec(memory_space=pltpu.SMEM),
        out_shape=jax.ShapeDtypeStruct((1, 1), jnp.float32),
    )(pacc.reshape(B * L // 128, 128), nacc.reshape(B * K * L // 128, 128))


def kernel(center, pos, neg, in_embed, out_embed):
    center = center.astype(jnp.int32)
    pos = pos.astype(jnp.int32)
    neg_flat = neg.astype(jnp.int32).reshape(B * K)
    pacc, nacc = _make_sc_scores()(center, pos, neg_flat, in_embed, out_embed)
    return _tc_loss(pacc, nacc)[0, 0]
